# baseline (speedup 1.0000x reference)
import functools

import jax
import jax.numpy as jnp
import numpy as np
from jax import lax
from jax.experimental import pallas as pl
from jax.experimental.pallas import tpu as pltpu
from jax.experimental.pallas import tpu_sc as plsc

_N = 6400000
_LANES = 128
_ROWS = _N // _LANES
_BLOCK_ROWS = 5000
_GRID = _ROWS // _BLOCK_ROWS

_KEEP_PROB = 0.9
_INV_KEEP = np.float32(1.0 / _KEEP_PROB)

_KD0 = 2247515013
_KD1 = 2545468385
_K0 = np.int32(np.uint32(_KD0))
_K1 = np.int32(np.uint32(_KD1))
_K2 = np.int32(np.uint32((_KD0 ^ _KD1 ^ 0x1BD11BDA) & 0xFFFFFFFF))
_KS = (_K0, _K1, _K2)
_ROTS = ((13, 15, 26, 6), (17, 29, 16, 24))
_THRESH = np.uint32(838861 << 9)


def _rotl(x, r):
    return lax.shift_left(x, np.int32(r)) | lax.shift_right_logical(
        x, np.int32(32 - r))


def _cipher_bits(idx):
    x0 = _K0
    x1 = idx + _K1
    for i in range(5):
        for r in _ROTS[i % 2]:
            x0 = x0 + x1
            x1 = _rotl(x1, r) ^ x0
        x0 = x0 + _KS[(i + 1) % 3]
        x1 = x1 + _KS[(i + 2) % 3] + np.int32(i + 1)
    return x0 ^ x1


def _dropout_block(v_ref, o_ref):
    pid = pl.program_id(0)
    base = pid * np.int32(_BLOCK_ROWS * _LANES)
    idx = (base
           + lax.broadcasted_iota(jnp.int32, (_BLOCK_ROWS, _LANES), 0)
           * np.int32(_LANES)
           + lax.broadcasted_iota(jnp.int32, (_BLOCK_ROWS, _LANES), 1))
    bits = _cipher_bits(idx)
    keep = lax.bitcast_convert_type(bits, jnp.uint32) >= _THRESH
    o_ref[...] = jnp.where(keep, v_ref[...] * _INV_KEEP, np.float32(0.0))


_SC_NW = 25
_SC_W_COLS = _N // _SC_NW
_SC_CH = 32000
_SC_NCH = _SC_W_COLS // _SC_CH


def _sc_copy_body(idx_hbm, out_hbm, buf, isems, osems):
    c = lax.axis_index("c")
    s = lax.axis_index("s")
    w = s * 2 + c

    @pl.when(w < _SC_NW)
    def _():
        col0 = w * _SC_W_COLS

        def cin(i):
            return pltpu.make_async_copy(
                idx_hbm.at[:, pl.ds(col0 + i * _SC_CH, _SC_CH)],
                buf.at[i % 2], isems.at[i % 2])

        def cout(i):
            return pltpu.make_async_copy(
                buf.at[i % 2],
                out_hbm.at[:, pl.ds(col0 + i * _SC_CH, _SC_CH)],
                osems.at[i % 2])

        cin(0).start()
        for i in range(_SC_NCH):
            cin(i).wait()
            cout(i).start()
            if i + 1 < _SC_NCH:
                if i >= 1:
                    cout(i - 1).wait()
                cin(i + 1).start()
        cout(_SC_NCH - 2).wait()
        cout(_SC_NCH - 1).wait()


def kernel(indices, values):
    sc_copy = functools.partial(
        pl.kernel,
        out_type=jax.ShapeDtypeStruct(indices.shape, indices.dtype),
        mesh=plsc.VectorSubcoreMesh(core_axis_name="c", subcore_axis_name="s"),
        scratch_types=[
            pltpu.VMEM((2, 2, _SC_CH), indices.dtype),
            pltpu.SemaphoreType.DMA((2,)),
            pltpu.SemaphoreType.DMA((2,)),
        ],
    )(_sc_copy_body)
    idx_out = sc_copy(indices)

    v2d = values.reshape(_ROWS, _LANES)
    out = pl.pallas_call(
        _dropout_block,
        grid=(_GRID,),
        in_specs=[pl.BlockSpec((_BLOCK_ROWS, _LANES), lambda i: (i, 0))],
        out_specs=pl.BlockSpec((_BLOCK_ROWS, _LANES), lambda i: (i, 0)),
        out_shape=jax.ShapeDtypeStruct((_ROWS, _LANES), jnp.float32),
    )(v2d)
    return idx_out, out.reshape(_N)


# speedup vs baseline: 1.0083x; 1.0083x over previous
import functools

import jax
import jax.numpy as jnp
import numpy as np
from jax import lax
from jax.experimental import pallas as pl
from jax.experimental.pallas import tpu as pltpu
from jax.experimental.pallas import tpu_sc as plsc

_N = 6400000
_LANES = 128
_ROWS = _N // _LANES
_BLOCK_ROWS = 2000
_GRID = _ROWS // _BLOCK_ROWS

_KEEP_PROB = 0.9
_INV_KEEP = np.float32(1.0 / _KEEP_PROB)

_KD0 = 2247515013
_KD1 = 2545468385
_K0 = np.int32(np.uint32(_KD0))
_K1 = np.int32(np.uint32(_KD1))
_K2 = np.int32(np.uint32((_KD0 ^ _KD1 ^ 0x1BD11BDA) & 0xFFFFFFFF))
_KS = (_K0, _K1, _K2)
_ROTS = ((13, 15, 26, 6), (17, 29, 16, 24))
_THRESH = np.uint32(838861 << 9)


def _rotl(x, r):
    return lax.shift_left(x, np.int32(r)) | lax.shift_right_logical(
        x, np.int32(32 - r))


def _cipher_bits(x1):
    x0 = _K0
    for i in range(5):
        for r in _ROTS[i % 2]:
            x0 = x0 + x1
            x1 = _rotl(x1, r) ^ x0
        x0 = x0 + _KS[(i + 1) % 3]
        x1 = x1 + _KS[(i + 2) % 3] + np.int32(i + 1)
    return x0 ^ x1


def _dropout_block(v_ref, o_ref):
    pid = pl.program_id(0)
    rowcol = (lax.broadcasted_iota(jnp.int32, (_BLOCK_ROWS, _LANES), 0)
              * np.int32(_LANES)
              + lax.broadcasted_iota(jnp.int32, (_BLOCK_ROWS, _LANES), 1))
    base_k1 = pid * np.int32(_BLOCK_ROWS * _LANES) + _K1
    bits = _cipher_bits(rowcol + base_k1)
    keep = lax.bitcast_convert_type(bits, jnp.uint32) >= _THRESH
    o_ref[...] = jnp.where(keep, v_ref[...] * _INV_KEEP, np.float32(0.0))


_SC_NW = 25
_SC_W_COLS = _N // _SC_NW
_SC_CH = 32000
_SC_NCH = _SC_W_COLS // _SC_CH


def _sc_copy_body(idx_hbm, out_hbm, buf, isems, osems):
    c = lax.axis_index("c")
    s = lax.axis_index("s")
    w = s * 2 + c

    @pl.when(w < _SC_NW)
    def _():
        col0 = w * _SC_W_COLS

        def cin(i):
            return pltpu.make_async_copy(
                idx_hbm.at[:, pl.ds(col0 + i * _SC_CH, _SC_CH)],
                buf.at[i % 2], isems.at[i % 2])

        def cout(i):
            return pltpu.make_async_copy(
                buf.at[i % 2],
                out_hbm.at[:, pl.ds(col0 + i * _SC_CH, _SC_CH)],
                osems.at[i % 2])

        cin(0).start()
        for i in range(_SC_NCH):
            cin(i).wait()
            cout(i).start()
            if i + 1 < _SC_NCH:
                if i >= 1:
                    cout(i - 1).wait()
                cin(i + 1).start()
        cout(_SC_NCH - 2).wait()
        cout(_SC_NCH - 1).wait()


def kernel(indices, values):
    sc_copy = functools.partial(
        pl.kernel,
        out_type=jax.ShapeDtypeStruct(indices.shape, indices.dtype),
        mesh=plsc.VectorSubcoreMesh(core_axis_name="c", subcore_axis_name="s"),
        scratch_types=[
            pltpu.VMEM((2, 2, _SC_CH), indices.dtype),
            pltpu.SemaphoreType.DMA((2,)),
            pltpu.SemaphoreType.DMA((2,)),
        ],
    )(_sc_copy_body)
    idx_out = sc_copy(indices)

    v2d = values.reshape(_ROWS, _LANES)
    out = pl.pallas_call(
        _dropout_block,
        grid=(_GRID,),
        in_specs=[pl.BlockSpec((_BLOCK_ROWS, _LANES), lambda i: (i, 0))],
        out_specs=pl.BlockSpec((_BLOCK_ROWS, _LANES), lambda i: (i, 0)),
        out_shape=jax.ShapeDtypeStruct((_ROWS, _LANES), jnp.float32),
    )(v2d)
    return idx_out, out.reshape(_N)
